# baseline (speedup 1.0000x reference)
import functools

import jax
import jax.numpy as jnp
from jax.experimental import pallas as pl
from jax.experimental.pallas import tpu as pltpu


def _qsample_body(t_ref, ab_ref, x0_ref, noise_ref, out_ref, *, nb):
    g = pl.program_id(0)
    for j in range(nb):
        b = g * nb + j
        ab = ab_ref[t_ref[b]]
        sa = jnp.sqrt(ab)
        sb = jnp.sqrt(1.0 - ab)
        out_ref[j] = sa * x0_ref[j] + sb * noise_ref[j]


@jax.jit
def kernel(x0, t, noise, alpha_bars):
    B, L, D = x0.shape
    NB = 4
    grid = (B // NB,)
    blk = pl.BlockSpec((NB, L, D), lambda g: (g, 0, 0))
    return pl.pallas_call(
        functools.partial(_qsample_body, nb=NB),
        grid=grid,
        in_specs=[
            pl.BlockSpec(memory_space=pltpu.SMEM),
            pl.BlockSpec(memory_space=pltpu.SMEM),
            blk,
            blk,
        ],
        out_specs=blk,
        out_shape=jax.ShapeDtypeStruct((B, L, D), jnp.float32),
        compiler_params=pltpu.CompilerParams(
            dimension_semantics=("arbitrary",),
        ),
    )(t, alpha_bars, x0, noise)


# speedup vs baseline: 1.0028x; 1.0028x over previous
import functools

import jax
import jax.numpy as jnp
from jax.experimental import pallas as pl
from jax.experimental.pallas import tpu as pltpu


def _qsample_body(t_ref, ab_ref, x0_ref, noise_ref, out_ref, *, nb):
    g = pl.program_id(0)
    for j in range(nb):
        b = g * nb + j
        ab = ab_ref[t_ref[b]]
        sa = jnp.sqrt(ab)
        sb = jnp.sqrt(1.0 - ab)
        out_ref[j] = sa * x0_ref[j] + sb * noise_ref[j]


@jax.jit
def kernel(x0, t, noise, alpha_bars):
    B, L, D = x0.shape
    NB = 4
    grid = (B // NB,)
    blk = pl.BlockSpec((NB, L, D), lambda g: (g, 0, 0))
    return pl.pallas_call(
        functools.partial(_qsample_body, nb=NB),
        grid=grid,
        in_specs=[
            pl.BlockSpec(memory_space=pltpu.SMEM),
            pl.BlockSpec(memory_space=pltpu.SMEM),
            blk,
            blk,
        ],
        out_specs=blk,
        out_shape=jax.ShapeDtypeStruct((B, L, D), jnp.float32),
        compiler_params=pltpu.CompilerParams(
            dimension_semantics=("parallel",),
        ),
    )(t, alpha_bars, x0, noise)
